# Initial kernel scaffold; baseline (speedup 1.0000x reference)
import jax
import jax.numpy as jnp
from jax.experimental import pallas as pl


def kernel(vertex_ids, vertex_embedding_weight):
    raise NotImplementedError("write your pallas kernel here")


# speedup vs baseline: 3.5400x; 3.5400x over previous
import functools

import jax
import jax.numpy as jnp
from jax import lax
from jax.experimental import pallas as pl
from jax.experimental.pallas import tpu as pltpu
from jax.experimental.pallas import tpu_sc as plsc

_ST_SIZE = 100000
_CHUNK = 128


def _build_gather(n_chunks_total, hidden, n_workers, nc):
    cpw = n_chunks_total // n_workers
    mesh = plsc.VectorSubcoreMesh(core_axis_name="c", subcore_axis_name="s")

    @functools.partial(
        pl.kernel,
        mesh=mesh,
        out_type=jax.ShapeDtypeStruct((n_chunks_total * _CHUNK, hidden),
                                      jnp.float32),
        scratch_types=[
            pltpu.VMEM((cpw, _CHUNK), jnp.int32),
            pltpu.VMEM((_CHUNK, hidden), jnp.float32),
            pltpu.SemaphoreType.DMA,
        ],
        compiler_params=pltpu.CompilerParams(use_tc_tiling_on_sc=False),
    )
    def gather_kernel(idx_hbm, table_hbm, out_hbm, idx_v, rows_v, sem):
        wid = lax.axis_index("s") * nc + lax.axis_index("c")
        base = wid * cpw
        pltpu.sync_copy(idx_hbm.at[pl.ds(base, cpw)], idx_v)

        def chunk(j, carry):
            pltpu.async_copy(table_hbm.at[idx_v.at[j]], rows_v, sem).wait()
            pltpu.sync_copy(rows_v,
                            out_hbm.at[pl.ds((base + j) * _CHUNK, _CHUNK)])
            return carry

        lax.fori_loop(0, cpw, chunk, 0)

    return gather_kernel


def kernel(vertex_ids, vertex_embedding_weight):
    b, l = vertex_ids.shape
    st, hidden = vertex_embedding_weight.shape
    n = b * l
    n_chunks = n // _CHUNK
    info = plsc.get_sparse_core_info()
    nc, ns = info.num_cores, info.num_subcores
    idx = vertex_ids.reshape(n_chunks, _CHUNK).astype(jnp.int32)
    fn = _build_gather(n_chunks, hidden, nc * ns, nc)
    out = fn(idx, vertex_embedding_weight)
    return out.reshape(b, l, hidden)


# speedup vs baseline: 4.2592x; 1.2032x over previous
import functools

import jax
import jax.numpy as jnp
from jax import lax
from jax.experimental import pallas as pl
from jax.experimental.pallas import tpu as pltpu
from jax.experimental.pallas import tpu_sc as plsc

_ST_SIZE = 100000
_CHUNK = 512
_NBUF = 2


def _build_gather(n_total, hidden, n_workers, nc):
    rpw = n_total // n_workers
    cpw = rpw // _CHUNK
    mesh = plsc.VectorSubcoreMesh(core_axis_name="c", subcore_axis_name="s")

    @functools.partial(
        pl.kernel,
        mesh=mesh,
        out_type=jax.ShapeDtypeStruct((n_total, hidden), jnp.float32),
        scratch_types=[
            pltpu.VMEM((cpw, _CHUNK), jnp.int32),
            pltpu.VMEM((_NBUF, _CHUNK, hidden), jnp.float32),
            pltpu.SemaphoreType.DMA((_NBUF,)),
            pltpu.SemaphoreType.DMA((_NBUF,)),
        ],
        compiler_params=pltpu.CompilerParams(use_tc_tiling_on_sc=False),
    )
    def gather_kernel(idx_hbm, table_hbm, out_hbm, idx_v, rows_v, gsem, osem):
        wid = lax.axis_index("s") * nc + lax.axis_index("c")
        base = wid * rpw
        pltpu.sync_copy(idx_hbm.at[pl.ds(wid * cpw, cpw)], idx_v)

        def start_gather(j, b):
            pltpu.async_copy(table_hbm.at[idx_v.at[j]],
                             rows_v.at[b], gsem.at[b])

        for b in range(_NBUF):
            start_gather(b, b)

        def step(j, carry):
            b = lax.rem(j, _NBUF)
            pltpu.make_async_copy(table_hbm.at[idx_v.at[0]],
                                  rows_v.at[b], gsem.at[b]).wait()
            out_slc = out_hbm.at[pl.ds(base + j * _CHUNK, _CHUNK)]
            pltpu.async_copy(rows_v.at[b], out_slc, osem.at[b])

            @pl.when(j + _NBUF < cpw)
            def _():
                pltpu.make_async_copy(rows_v.at[b], out_slc,
                                      osem.at[b]).wait()
                start_gather(j + _NBUF, b)

            return carry

        lax.fori_loop(0, cpw, step, 0)

        def drain(j, carry):
            b = lax.rem(j, _NBUF)
            pltpu.make_async_copy(
                rows_v.at[b],
                out_hbm.at[pl.ds(base, _CHUNK)], osem.at[b]).wait()
            return carry

        lax.fori_loop(cpw - _NBUF, cpw, drain, 0)

    return gather_kernel


def kernel(vertex_ids, vertex_embedding_weight):
    b, l = vertex_ids.shape
    st, hidden = vertex_embedding_weight.shape
    n = b * l
    info = plsc.get_sparse_core_info()
    nc, ns = info.num_cores, info.num_subcores
    idx = vertex_ids.reshape(n // _CHUNK, _CHUNK).astype(jnp.int32)
    fn = _build_gather(n, hidden, nc * ns, nc)
    out = fn(idx, vertex_embedding_weight)
    return out.reshape(b, l, hidden)


# speedup vs baseline: 4.2696x; 1.0024x over previous
import functools

import jax
import jax.numpy as jnp
from jax import lax
from jax.experimental import pallas as pl
from jax.experimental.pallas import tpu as pltpu
from jax.experimental.pallas import tpu_sc as plsc

_ST_SIZE = 100000
_CHUNK = 512
_NBUF = 3
_LOOKAHEAD = 2


def _build_gather(n_total, hidden, n_workers, nc):
    rpw = n_total // n_workers
    cpw = rpw // _CHUNK
    mesh = plsc.VectorSubcoreMesh(core_axis_name="c", subcore_axis_name="s")

    @functools.partial(
        pl.kernel,
        mesh=mesh,
        out_type=jax.ShapeDtypeStruct((n_total, hidden), jnp.float32),
        scratch_types=[
            pltpu.VMEM((cpw, _CHUNK), jnp.int32),
            pltpu.VMEM((_NBUF, _CHUNK, hidden), jnp.float32),
            pltpu.SemaphoreType.DMA((_NBUF,)),
            pltpu.SemaphoreType.DMA((_NBUF,)),
        ],
        compiler_params=pltpu.CompilerParams(use_tc_tiling_on_sc=False),
    )
    def gather_kernel(idx_hbm, table_hbm, out_hbm, idx_v, rows_v, gsem, osem):
        wid = lax.axis_index("s") * nc + lax.axis_index("c")
        base = wid * rpw
        pltpu.sync_copy(idx_hbm.at[pl.ds(wid * cpw, cpw)], idx_v)

        def step(t, carry):
            @pl.when(t < cpw)
            def _():
                b = lax.rem(t, _NBUF)

                @pl.when(t >= _NBUF)
                def _():
                    pltpu.make_async_copy(
                        rows_v.at[b],
                        out_hbm.at[pl.ds(base, _CHUNK)], osem.at[b]).wait()

                pltpu.async_copy(table_hbm.at[idx_v.at[t]],
                                 rows_v.at[b], gsem.at[b])

            d = t - _LOOKAHEAD

            @pl.when(jnp.logical_and(d >= 0, d < cpw))
            def _():
                bd = lax.rem(d, _NBUF)
                pltpu.make_async_copy(table_hbm.at[idx_v.at[0]],
                                      rows_v.at[bd], gsem.at[bd]).wait()
                pltpu.async_copy(
                    rows_v.at[bd],
                    out_hbm.at[pl.ds(base + d * _CHUNK, _CHUNK)],
                    osem.at[bd])

            return carry

        lax.fori_loop(0, cpw + _LOOKAHEAD, step, 0)

        def drain(j, carry):
            b = lax.rem(j, _NBUF)
            pltpu.make_async_copy(
                rows_v.at[b],
                out_hbm.at[pl.ds(base, _CHUNK)], osem.at[b]).wait()
            return carry

        lax.fori_loop(cpw - _NBUF, cpw, drain, 0)

    return gather_kernel


def kernel(vertex_ids, vertex_embedding_weight):
    b, l = vertex_ids.shape
    st, hidden = vertex_embedding_weight.shape
    n = b * l
    info = plsc.get_sparse_core_info()
    nc, ns = info.num_cores, info.num_subcores
    idx = vertex_ids.reshape(n // _CHUNK, _CHUNK).astype(jnp.int32)
    fn = _build_gather(n, hidden, nc * ns, nc)
    out = fn(idx, vertex_embedding_weight)
    return out.reshape(b, l, hidden)
